# Initial kernel scaffold; baseline (speedup 1.0000x reference)
import jax
import jax.numpy as jnp
from jax.experimental import pallas as pl


def kernel(feats, coords, kernel):
    raise NotImplementedError("write your pallas kernel here")


# speedup vs baseline: 6.0899x; 6.0899x over previous
import functools

import jax
import jax.numpy as jnp
from jax import lax
from jax.experimental import pallas as pl
from jax.experimental.pallas import tpu as pltpu
from jax.experimental.pallas import tpu_sc as plsc

N = 50000
CIN = 32
COUT = 32
KV = 27
G = 70
R = 343040
DUMP = 343000
NP = 50176
NPAD = 50008
NQ = NP * KV
NW = 32

ROWS_PER_TILE = R // NW
BUILD_CHUNK = 1072
Q_PER_TILE = NQ // NW
Q_CHUNK = 2016

_mesh = plsc.VectorSubcoreMesh(core_axis_name="c", subcore_axis_name="s")
_sc_params = pltpu.CompilerParams(use_tc_tiling_on_sc=False)


@functools.partial(
    pl.kernel,
    out_type=jax.ShapeDtypeStruct((R, CIN), jnp.float32),
    mesh=_mesh,
    compiler_params=_sc_params,
    scratch_types=[
        pltpu.VMEM((BUILD_CHUNK,), jnp.int32),
        pltpu.VMEM((BUILD_CHUNK, CIN), jnp.float32),
        pltpu.SemaphoreType.DMA,
    ],
)
def _build_table(gridmin_hbm, feats_hbm, table_hbm, idx_v, rows_v, sem):
    wid = lax.axis_index("s") * 2 + lax.axis_index("c")
    base = wid * ROWS_PER_TILE

    @pl.loop(0, ROWS_PER_TILE, step=BUILD_CHUNK)
    def _(off):
        pltpu.sync_copy(gridmin_hbm.at[pl.ds(base + off, BUILD_CHUNK)], idx_v)
        pltpu.async_copy(feats_hbm.at[idx_v], rows_v, sem).wait()
        pltpu.sync_copy(rows_v, table_hbm.at[pl.ds(base + off, BUILD_CHUNK)])


@functools.partial(
    pl.kernel,
    out_type=jax.ShapeDtypeStruct((NQ, CIN), jnp.float32),
    mesh=_mesh,
    compiler_params=_sc_params,
    scratch_types=[
        pltpu.VMEM((Q_CHUNK,), jnp.int32),
        pltpu.VMEM((Q_CHUNK, CIN), jnp.float32),
        pltpu.SemaphoreType.DMA,
    ],
)
def _gather_rows(q_hbm, table_hbm, out_hbm, idx_v, rows_v, sem):
    wid = lax.axis_index("s") * 2 + lax.axis_index("c")
    base = wid * Q_PER_TILE

    @pl.loop(0, Q_PER_TILE, step=Q_CHUNK)
    def _(off):
        pltpu.sync_copy(q_hbm.at[pl.ds(base + off, Q_CHUNK)], idx_v)
        pltpu.async_copy(table_hbm.at[idx_v], rows_v, sem).wait()
        pltpu.sync_copy(rows_v, out_hbm.at[pl.ds(base + off, Q_CHUNK)])


BLK = 512


def _mm_body(g_ref, w_ref, o_ref):
    o_ref[...] = jnp.dot(g_ref[...], w_ref[...],
                         preferred_element_type=jnp.float32)


def _matmul(gathered, wflat):
    return pl.pallas_call(
        _mm_body,
        grid=(NP // BLK,),
        in_specs=[
            pl.BlockSpec((BLK, KV * CIN), lambda i: (i, 0)),
            pl.BlockSpec((KV * CIN, COUT), lambda i: (0, 0)),
        ],
        out_specs=pl.BlockSpec((BLK, COUT), lambda i: (i, 0)),
        out_shape=jax.ShapeDtypeStruct((NP, COUT), jnp.float32),
    )(gathered, wflat)


_OFFS = [(dx * G + dy) * G + dz
         for dx in range(-1, 2) for dy in range(-1, 2) for dz in range(-1, 2)]


def kernel(feats, coords, kernel):
    w = kernel
    c = coords.astype(jnp.int32) + 1
    keys = (c[:, 0] * G + c[:, 1]) * G + c[:, 2]
    iota = jnp.arange(N, dtype=jnp.int32)
    gridmin = jnp.full((R,), N, jnp.int32).at[keys].min(iota)
    offs = jnp.array(_OFFS, dtype=jnp.int32)
    q = keys[:, None] + offs[None, :]
    q = jnp.concatenate([q, jnp.full((NP - N, KV), DUMP, jnp.int32)], axis=0)
    q = q.reshape(NQ)
    feats_pad = jnp.concatenate(
        [feats, jnp.zeros((NPAD - N, CIN), feats.dtype)], axis=0)
    table = _build_table(gridmin, feats_pad)
    gathered = _gather_rows(q, table)
    out = _matmul(gathered.reshape(NP, KV * CIN), w.reshape(KV * CIN, COUT))
    return out[:N]


# speedup vs baseline: 10.9225x; 1.7936x over previous
import functools

import jax
import jax.numpy as jnp
from jax import lax
from jax.experimental import pallas as pl
from jax.experimental.pallas import tpu as pltpu
from jax.experimental.pallas import tpu_sc as plsc

N = 50000
CIN = 32
COUT = 32
KV = 27
G = 70
R = 343040
DUMP = 343000
NP = 50176
NPAD = 50008
NQ = NP * KV
NW = 32

ROWS_PER_TILE = R // NW
BUILD_CHUNK = 1072
Q_PER_TILE = NQ // NW
Q_CHUNK = 1008

_mesh = plsc.VectorSubcoreMesh(core_axis_name="c", subcore_axis_name="s")
_sc_params = pltpu.CompilerParams(use_tc_tiling_on_sc=False)


@functools.partial(
    pl.kernel,
    out_type=jax.ShapeDtypeStruct((R, CIN), jnp.float32),
    mesh=_mesh,
    compiler_params=_sc_params,
    scratch_types=[],
)
def _build_table(gridmin_hbm, feats_hbm, table_hbm):
    def body(i_vmem, o_vmem):
        pltpu.sync_copy(feats_hbm.at[i_vmem.at[0]], o_vmem)

    pltpu.emit_pipeline(
        body,
        grid=(R // BUILD_CHUNK,),
        in_specs=[pl.BlockSpec((1, BUILD_CHUNK), lambda i: (0, i))],
        out_specs=[pl.BlockSpec((BUILD_CHUNK, CIN), lambda i: (i, 0))],
        core_axis_name=("c", "s"),
        dimension_semantics=(pltpu.PARALLEL,),
    )(gridmin_hbm, table_hbm)


@functools.partial(
    pl.kernel,
    out_type=jax.ShapeDtypeStruct((NQ, CIN), jnp.float32),
    mesh=_mesh,
    compiler_params=_sc_params,
    scratch_types=[],
)
def _gather_rows(q_hbm, table_hbm, out_hbm):
    def body(i_vmem, o_vmem):
        pltpu.sync_copy(table_hbm.at[i_vmem.at[0]], o_vmem)

    pltpu.emit_pipeline(
        body,
        grid=(NQ // Q_CHUNK,),
        in_specs=[pl.BlockSpec((1, Q_CHUNK), lambda i: (0, i))],
        out_specs=[pl.BlockSpec((Q_CHUNK, CIN), lambda i: (i, 0))],
        core_axis_name=("c", "s"),
        dimension_semantics=(pltpu.PARALLEL,),
    )(q_hbm, out_hbm)


BLK = 512


def _mm_body(g_ref, w_ref, o_ref):
    o_ref[...] = jnp.dot(g_ref[...], w_ref[...],
                         preferred_element_type=jnp.float32)


def _matmul(gathered, wflat):
    return pl.pallas_call(
        _mm_body,
        grid=(NP // BLK,),
        in_specs=[
            pl.BlockSpec((BLK, KV * CIN), lambda i: (i, 0)),
            pl.BlockSpec((KV * CIN, COUT), lambda i: (0, 0)),
        ],
        out_specs=pl.BlockSpec((BLK, COUT), lambda i: (i, 0)),
        out_shape=jax.ShapeDtypeStruct((NP, COUT), jnp.float32),
    )(gathered, wflat)


_OFFS = [(dx * G + dy) * G + dz
         for dx in range(-1, 2) for dy in range(-1, 2) for dz in range(-1, 2)]


def kernel(feats, coords, kernel):
    w = kernel
    c = coords.astype(jnp.int32) + 1
    keys = (c[:, 0] * G + c[:, 1]) * G + c[:, 2]
    iota = jnp.arange(N, dtype=jnp.int32)
    gridmin = jnp.full((R,), N, jnp.int32).at[keys].min(iota)
    offs = jnp.array(_OFFS, dtype=jnp.int32)
    q = keys[:, None] + offs[None, :]
    q = jnp.concatenate([q, jnp.full((NP - N, KV), DUMP, jnp.int32)], axis=0)
    q = q.reshape(NQ)
    feats_pad = jnp.concatenate(
        [feats, jnp.zeros((NPAD - N, CIN), feats.dtype)], axis=0)
    table = _build_table(gridmin.reshape(1, R), feats_pad)
    gathered = _gather_rows(q.reshape(1, NQ), table)
    out = _matmul(gathered.reshape(NP, KV * CIN), w.reshape(KV * CIN, COUT))
    return out[:N]
